# baseline (speedup 1.0000x reference)
import jax
import jax.numpy as jnp
from jax import lax
from jax.experimental import pallas as pl
from jax.experimental.pallas import tpu as pltpu
from jax.experimental.pallas import tpu_sc as plsc

N = 10000
E = 160000
D = 256
HALF = 128
CLS = 16
NC = 2
NS = 16
L = 16

NPAD = 10240
CK = 112
CH1 = 96
EPAD = NS * CH1 * CK
CH2 = EPAD // (NC * NS) // CK
NACC = 10112
RPT = NACC // NS

BN = 1024
NB = NPAD // BN


def _mesh():
    return plsc.VectorSubcoreMesh(core_axis_name="c", subcore_axis_name="s")


def _deg_body(dst_hbm, degp_hbm, idxv, dloc2, iotar, dsh):
    c = lax.axis_index("c")
    s = lax.axis_index("s")
    pltpu.sync_copy(dst_hbm.at[c, s], idxv)
    zeros16 = jnp.zeros((L,), jnp.float32)

    def zero_body(i, carry):
        dloc2[i // 8, pl.ds((i % 8) * L, L)] = zeros16
        return carry

    lax.fori_loop(0, NPAD // L, zero_body, 0)
    iota16 = lax.iota(jnp.int32, L)

    def iota_body(q, carry):
        iotar[pl.ds(q * L, L)] = iota16 + q * L
        return carry

    lax.fori_loop(0, (NPAD // 128) // L, iota_body, 0)

    @pl.when(s == 0)
    def _():
        pltpu.sync_copy(dloc2, dsh)

    plsc.subcore_barrier()
    ones16 = jnp.ones((L,), jnp.float32)
    KV = CK // L

    def hist_body(i, carry):
        j = i // KV
        k = i % KV
        idx = idxv[j, pl.ds(k * L, L)]
        plsc.addupdate_scatter(dloc2, [idx >> 7, idx & 127], ones16)
        return carry

    lax.fori_loop(0, CH2 * KV, hist_body, 0)
    pltpu.sync_copy(dloc2, dsh.at[iotar], add=True)
    plsc.subcore_barrier()

    @pl.when(s == 0)
    def _():
        pltpu.sync_copy(dsh, degp_hbm.at[c])


def _deg_call(dst2):
    return pl.kernel(
        _deg_body,
        out_type=jax.ShapeDtypeStruct((NC, NPAD // 128, 128), jnp.float32),
        mesh=_mesh(),
        compiler_params=pltpu.CompilerParams(needs_layout_passes=False),
        scratch_types=[
            pltpu.VMEM((CH2, CK), jnp.int32),
            pltpu.VMEM((NPAD // 128, 128), jnp.float32),
            pltpu.VMEM((NPAD // 128,), jnp.int32),
            pltpu.VMEM_SHARED((NPAD // 128, 128), jnp.float32),
        ],
    )(dst2)


def _g1_body(x_ref, w1_ref, dinv_ref, g_ref):
    dinv = dinv_ref[...]
    h = jnp.dot(x_ref[...].astype(jnp.bfloat16), w1_ref[...].astype(jnp.bfloat16),
                preferred_element_type=jnp.float32)
    g = h * dinv
    g_ref[0] = g[:, :HALF]
    g_ref[1] = g[:, HALF:]


def _g1_call(x_pad, W1, dinv):
    return pl.pallas_call(
        _g1_body,
        grid=(NB,),
        in_specs=[
            pl.BlockSpec((BN, D), lambda i: (i, 0)),
            pl.BlockSpec((D, D), lambda i: (0, 0)),
            pl.BlockSpec((BN, 1), lambda i: (i, 0)),
        ],
        out_specs=pl.BlockSpec((NC, BN, HALF), lambda i: (0, i, 0)),
        out_shape=jax.ShapeDtypeStruct((NC, NPAD, HALF), jnp.float32),
    )(x_pad, W1, dinv)


GROUP = 24
NGRP = CH1 // GROUP


def _zero_tile_share(buf, acc, s, width):
    zeros16 = jnp.zeros((L,), jnp.float32)
    kv = width // L

    def zb(i, carry):
        buf[i // kv, pl.ds((i % kv) * L, L)] = zeros16
        return carry

    lax.fori_loop(0, CK * kv, zb, 0)
    for q in range(RPT // CK):
        pltpu.sync_copy(buf, acc.at[pl.ds(s * RPT + q * CK, CK)])
    rem = RPT - (RPT // CK) * CK
    if rem:
        pltpu.sync_copy(buf.at[pl.ds(0, rem)],
                        acc.at[pl.ds(s * RPT + (RPT // CK) * CK, rem)])


def _edge_pass(g_hbm, acc, srcv, dstv, r0, r1, r2,
               sg0, sg1, sg2, ss0, ss1, ss2, nch):
    def gath(j, buf, sem):
        pltpu.async_copy(g_hbm.at[srcv.at[j]], buf, sem)

    def gath_wait(j, buf, sem):
        pltpu.make_async_copy(g_hbm.at[srcv.at[j]], buf, sem).wait()

    def scat(j, buf, sem):
        pltpu.async_copy(buf, acc.at[dstv.at[j]], sem, add=True)

    def scat_wait(j, buf, sem):
        pltpu.make_async_copy(buf, acc.at[dstv.at[j]], sem).wait()

    gath(0, r0, sg0)
    gath(1, r1, sg1)

    def step(j2, carry):
        j = j2 * 3
        gath_wait(j, r0, sg0)
        scat(j, r0, ss0)

        @pl.when(j2 > 0)
        def _():
            scat_wait(j - 1, r2, ss2)

        gath(j + 2, r2, sg2)

        gath_wait(j + 1, r1, sg1)
        scat(j + 1, r1, ss1)

        @pl.when(j + 3 < nch)
        def _():
            scat_wait(j, r0, ss0)
            gath(j + 3, r0, sg0)

        gath_wait(j + 2, r2, sg2)
        scat(j + 2, r2, ss2)

        @pl.when(j + 4 < nch)
        def _():
            scat_wait(j + 1, r1, ss1)
            gath(j + 4, r1, sg1)

        return carry

    lax.fori_loop(0, nch // 3, step, 0)
    scat_wait(nch - 3, r0, ss0)
    scat_wait(nch - 2, r1, ss1)
    scat_wait(nch - 1, r2, ss2)


def _scat1_body(g_hbm, src_hbm, dst_hbm, out_hbm,
                srcv, dstv, r0, r1, r2, acc, sg0, sg1, sg2, ss0, ss1, ss2):
    c = lax.axis_index("c")
    s = lax.axis_index("s")
    _zero_tile_share(r0, acc, s, HALF)
    plsc.subcore_barrier()

    def group(gi, carry):
        pltpu.sync_copy(src_hbm.at[c, s, pl.ds(gi * GROUP, GROUP)], srcv)
        pltpu.sync_copy(dst_hbm.at[s, pl.ds(gi * GROUP, GROUP)], dstv)
        _edge_pass(g_hbm, acc, srcv, dstv, r0, r1, r2,
                   sg0, sg1, sg2, ss0, ss1, ss2, GROUP)
        return carry

    lax.fori_loop(0, NGRP, group, 0)
    plsc.subcore_barrier()
    pltpu.sync_copy(acc.at[pl.ds(s * RPT, RPT)],
                    out_hbm.at[c, pl.ds(s * RPT, RPT)])


def _scat1_call(g_flat, src1o, dst1):
    return pl.kernel(
        _scat1_body,
        out_type=jax.ShapeDtypeStruct((NC, NPAD, HALF), jnp.float32),
        mesh=_mesh(),
        scratch_types=[
            pltpu.VMEM((GROUP, CK), jnp.int32),
            pltpu.VMEM((GROUP, CK), jnp.int32),
            pltpu.VMEM((CK, HALF), jnp.float32),
            pltpu.VMEM((CK, HALF), jnp.float32),
            pltpu.VMEM((CK, HALF), jnp.float32),
            pltpu.VMEM_SHARED((NACC, HALF), jnp.float32),
            pltpu.SemaphoreType.DMA,
            pltpu.SemaphoreType.DMA,
            pltpu.SemaphoreType.DMA,
            pltpu.SemaphoreType.DMA,
            pltpu.SemaphoreType.DMA,
            pltpu.SemaphoreType.DMA,
        ],
    )(g_flat, src1o, dst1)


def _h2_body(scat_ref, ga_ref, gb_ref, dinv_ref, b1_ref, w2_ref, g2_ref):
    sc = scat_ref[...]
    h = jnp.concatenate([sc[0] + ga_ref[...], sc[1] + gb_ref[...]], axis=1)
    dinv = dinv_ref[...]
    o1 = jnp.maximum(h * dinv + b1_ref[...], 0.0)
    g2_ref[...] = jnp.dot(o1, w2_ref[...], preferred_element_type=jnp.float32) * dinv


def _h2_call(scat, g_flat, dinv, b1r, W2):
    return pl.pallas_call(
        _h2_body,
        grid=(NB,),
        in_specs=[
            pl.BlockSpec((NC, BN, HALF), lambda i: (0, i, 0)),
            pl.BlockSpec((BN, HALF), lambda i: (i, 0)),
            pl.BlockSpec((BN, HALF), lambda i: (NB + i, 0)),
            pl.BlockSpec((BN, 1), lambda i: (i, 0)),
            pl.BlockSpec((1, D), lambda i: (0, 0)),
            pl.BlockSpec((D, CLS), lambda i: (0, 0)),
        ],
        out_specs=pl.BlockSpec((BN, CLS), lambda i: (i, 0)),
        out_shape=jax.ShapeDtypeStruct((NPAD, CLS), jnp.float32),
    )(scat, g_flat, g_flat, dinv, b1r, W2)


def _scat2_body(g2_hbm, src_hbm, dst_hbm, out_hbm,
                srcv, dstv, r0, r1, r2, acc, sg0, sg1, sg2, ss0, ss1, ss2):
    c = lax.axis_index("c")
    s = lax.axis_index("s")
    _zero_tile_share(r0, acc, s, CLS)
    pltpu.sync_copy(src_hbm.at[c, s], srcv)
    pltpu.sync_copy(dst_hbm.at[c, s], dstv)
    plsc.subcore_barrier()
    _edge_pass(g2_hbm, acc, srcv, dstv, r0, r1, r2,
               sg0, sg1, sg2, ss0, ss1, ss2, CH2)
    plsc.subcore_barrier()
    pltpu.sync_copy(acc.at[pl.ds(s * RPT, RPT)],
                    out_hbm.at[c, pl.ds(s * RPT, RPT)])


def _scat2_call(g2, src2, dst2):
    return pl.kernel(
        _scat2_body,
        out_type=jax.ShapeDtypeStruct((NC, NPAD, CLS), jnp.float32),
        mesh=_mesh(),
        compiler_params=pltpu.CompilerParams(use_tc_tiling_on_sc=False),
        scratch_types=[
            pltpu.VMEM((CH2, CK), jnp.int32),
            pltpu.VMEM((CH2, CK), jnp.int32),
            pltpu.VMEM((CK, CLS), jnp.float32),
            pltpu.VMEM((CK, CLS), jnp.float32),
            pltpu.VMEM((CK, CLS), jnp.float32),
            pltpu.VMEM_SHARED((NACC, CLS), jnp.float32),
            pltpu.SemaphoreType.DMA,
            pltpu.SemaphoreType.DMA,
            pltpu.SemaphoreType.DMA,
            pltpu.SemaphoreType.DMA,
            pltpu.SemaphoreType.DMA,
            pltpu.SemaphoreType.DMA,
        ],
    )(g2, src2, dst2)


def _fin_body(p2_ref, g2_ref, dinv_ref, b2_ref, out_ref):
    agg = jnp.sum(p2_ref[...], axis=0) + g2_ref[...]
    out_ref[...] = agg * dinv_ref[...] + b2_ref[...]


FBN = 1000


def _fin_call(p2, g2, dinv, b2r):
    return pl.pallas_call(
        _fin_body,
        grid=(N // FBN,),
        in_specs=[
            pl.BlockSpec((NC, FBN, CLS), lambda i: (0, i, 0)),
            pl.BlockSpec((FBN, CLS), lambda i: (i, 0)),
            pl.BlockSpec((FBN, 1), lambda i: (i, 0)),
            pl.BlockSpec((1, CLS), lambda i: (0, 0)),
        ],
        out_specs=pl.BlockSpec((FBN, CLS), lambda i: (i, 0)),
        out_shape=jax.ShapeDtypeStruct((N, CLS), jnp.float32),
    )(p2, g2, dinv, b2r)


def kernel(x, edge_index, W1, b1, W2, b2):
    src = edge_index[0].astype(jnp.int32)
    dst = edge_index[1].astype(jnp.int32)
    pad_e = EPAD - E
    src_p = jnp.concatenate([src, jnp.zeros((pad_e,), jnp.int32)])
    dst_p = jnp.concatenate([dst, jnp.full((pad_e,), N, jnp.int32)])
    src1 = src_p.reshape(NS, CH1, CK)
    src1o = jnp.stack([src1, src1 + NPAD])
    dst1 = dst_p.reshape(NS, CH1, CK)
    src2 = src_p.reshape(NC, NS, CH2, CK)
    dst2 = dst_p.reshape(NC, NS, CH2, CK)
    x_pad = jnp.pad(x, ((0, NPAD - N), (0, 0)))
    b1r = b1.reshape(1, D)
    b2r = b2.reshape(1, CLS)

    degp = _deg_call(dst2)
    dinv = lax.rsqrt(degp[0] + degp[1] + 1.0).reshape(NPAD, 1)
    g3 = _g1_call(x_pad, W1, dinv)
    g_flat = g3.reshape(NC * NPAD, HALF)
    scat = _scat1_call(g_flat, src1o, dst1)
    g2 = _h2_call(scat, g_flat, dinv, b1r, W2)
    p2 = _scat2_call(g2, src2, dst2)
    return _fin_call(p2, g2, dinv, b2r)


# speedup vs baseline: 1.8701x; 1.8701x over previous
import jax
import jax.numpy as jnp
from jax import lax
from jax.experimental import pallas as pl
from jax.experimental.pallas import tpu as pltpu
from jax.experimental.pallas import tpu_sc as plsc

N = 10000
E = 160000
D = 256
HALF = 128
CLS = 16
NC = 2
NS = 16
L = 16

NPAD = 10240
CK = 128
CH1 = 80
EPAD = NS * CH1 * CK
CH2 = EPAD // (NC * NS) // CK
NACC = 10240
RPT = NACC // NS

BN = 1024
NB = NPAD // BN


def _mesh():
    return plsc.VectorSubcoreMesh(core_axis_name="c", subcore_axis_name="s")


def _deg_body(dst_hbm, degp_hbm, idxv, dloc2, iotar, dsh):
    c = lax.axis_index("c")
    s = lax.axis_index("s")
    pltpu.sync_copy(dst_hbm.at[c, s], idxv)
    zeros16 = jnp.zeros((L,), jnp.float32)

    def zero_body(i, carry):
        dloc2[i // 8, pl.ds((i % 8) * L, L)] = zeros16
        return carry

    lax.fori_loop(0, NPAD // L, zero_body, 0)
    iota16 = lax.iota(jnp.int32, L)

    def iota_body(q, carry):
        iotar[pl.ds(q * L, L)] = iota16 + q * L
        return carry

    lax.fori_loop(0, (NPAD // 128) // L, iota_body, 0)

    @pl.when(s == 0)
    def _():
        pltpu.sync_copy(dloc2, dsh)

    plsc.subcore_barrier()
    ones16 = jnp.ones((L,), jnp.float32)
    KV = CK // L

    def hist_body(i, carry):
        j = i // KV
        k = i % KV
        idx = idxv[j, pl.ds(k * L, L)]
        plsc.addupdate_scatter(dloc2, [idx >> 7, idx & 127], ones16)
        return carry

    lax.fori_loop(0, CH2 * KV, hist_body, 0)
    pltpu.sync_copy(dloc2, dsh.at[iotar], add=True)
    plsc.subcore_barrier()

    @pl.when(s == 0)
    def _():
        pltpu.sync_copy(dsh, degp_hbm.at[c])


def _deg_call(dst2):
    return pl.kernel(
        _deg_body,
        out_type=jax.ShapeDtypeStruct((NC, NPAD // 128, 128), jnp.float32),
        mesh=_mesh(),
        compiler_params=pltpu.CompilerParams(needs_layout_passes=False),
        scratch_types=[
            pltpu.VMEM((CH2, CK), jnp.int32),
            pltpu.VMEM((NPAD // 128, 128), jnp.float32),
            pltpu.VMEM((NPAD // 128,), jnp.int32),
            pltpu.VMEM_SHARED((NPAD // 128, 128), jnp.float32),
        ],
    )(dst2)


def _g1_body(x_ref, w1_ref, dinv_ref, g_ref):
    dinv = dinv_ref[...]
    h = jnp.dot(x_ref[...].astype(jnp.bfloat16), w1_ref[...].astype(jnp.bfloat16),
                preferred_element_type=jnp.float32)
    g = h * dinv
    g_ref[0] = g[:, :HALF]
    g_ref[1] = g[:, HALF:]


def _g1_call(x_pad, W1, dinv):
    return pl.pallas_call(
        _g1_body,
        grid=(NB,),
        in_specs=[
            pl.BlockSpec((BN, D), lambda i: (i, 0)),
            pl.BlockSpec((D, D), lambda i: (0, 0)),
            pl.BlockSpec((BN, 1), lambda i: (i, 0)),
        ],
        out_specs=pl.BlockSpec((NC, BN, HALF), lambda i: (0, i, 0)),
        out_shape=jax.ShapeDtypeStruct((NC, NPAD, HALF), jnp.float32),
    )(x_pad, W1, dinv)


GROUP = 16
NGRP = CH1 // GROUP


def _zero_tile_share(buf, acc, s, width):
    zeros16 = jnp.zeros((L,), jnp.float32)
    kv = width // L

    def zb(i, carry):
        buf[i // kv, pl.ds((i % kv) * L, L)] = zeros16
        return carry

    lax.fori_loop(0, CK * kv, zb, 0)
    for q in range(RPT // CK):
        pltpu.sync_copy(buf, acc.at[pl.ds(s * RPT + q * CK, CK)])


def _edge_pass(g_hbm, acc, srcv, dstv, r0, r1, sem0, sem1, nch):
    pltpu.async_copy(g_hbm.at[srcv.at[0]], r0, sem0)

    def step(j2, carry):
        j = j2 * 2
        pltpu.make_async_copy(g_hbm.at[srcv.at[j]], r0, sem0).wait()
        pltpu.async_copy(g_hbm.at[srcv.at[j + 1]], r1, sem1)
        pltpu.sync_copy(r0, acc.at[dstv.at[j]], add=True)
        pltpu.make_async_copy(g_hbm.at[srcv.at[j + 1]], r1, sem1).wait()

        @pl.when(j + 2 < nch)
        def _():
            pltpu.async_copy(g_hbm.at[srcv.at[j + 2]], r0, sem0)

        pltpu.sync_copy(r1, acc.at[dstv.at[j + 1]], add=True)
        return carry

    lax.fori_loop(0, nch // 2, step, 0)


def _scat1_body(g_hbm, src_hbm, dst_hbm, out_hbm,
                srcv, dstv, r0, r1, acc, sem0, sem1):
    c = lax.axis_index("c")
    s = lax.axis_index("s")
    _zero_tile_share(r0, acc, s, HALF)
    plsc.subcore_barrier()

    def group(gi, carry):
        pltpu.sync_copy(src_hbm.at[c, s, pl.ds(gi * GROUP, GROUP)], srcv)
        pltpu.sync_copy(dst_hbm.at[s, pl.ds(gi * GROUP, GROUP)], dstv)
        _edge_pass(g_hbm, acc, srcv, dstv, r0, r1, sem0, sem1, GROUP)
        return carry

    lax.fori_loop(0, NGRP, group, 0)
    plsc.subcore_barrier()
    pltpu.sync_copy(acc.at[pl.ds(s * RPT, RPT)],
                    out_hbm.at[c, pl.ds(s * RPT, RPT)])


def _scat1_call(g_flat, src1o, dst1):
    return pl.kernel(
        _scat1_body,
        out_type=jax.ShapeDtypeStruct((NC, NPAD, HALF), jnp.float32),
        mesh=_mesh(),
        scratch_types=[
            pltpu.VMEM((GROUP, CK), jnp.int32),
            pltpu.VMEM((GROUP, CK), jnp.int32),
            pltpu.VMEM((CK, HALF), jnp.float32),
            pltpu.VMEM((CK, HALF), jnp.float32),
            pltpu.VMEM_SHARED((NACC, HALF), jnp.float32),
            pltpu.SemaphoreType.DMA,
            pltpu.SemaphoreType.DMA,
        ],
    )(g_flat, src1o, dst1)


def _h2_body(scat_ref, ga_ref, gb_ref, dinv_ref, b1_ref, w2_ref, g2_ref):
    sc = scat_ref[...]
    h = jnp.concatenate([sc[0] + ga_ref[...], sc[1] + gb_ref[...]], axis=1)
    dinv = dinv_ref[...]
    o1 = jnp.maximum(h * dinv + b1_ref[...], 0.0)
    g2_ref[...] = jnp.dot(o1, w2_ref[...], preferred_element_type=jnp.float32) * dinv


def _h2_call(scat, g_flat, dinv, b1r, W2):
    return pl.pallas_call(
        _h2_body,
        grid=(NB,),
        in_specs=[
            pl.BlockSpec((NC, BN, HALF), lambda i: (0, i, 0)),
            pl.BlockSpec((BN, HALF), lambda i: (i, 0)),
            pl.BlockSpec((BN, HALF), lambda i: (NB + i, 0)),
            pl.BlockSpec((BN, 1), lambda i: (i, 0)),
            pl.BlockSpec((1, D), lambda i: (0, 0)),
            pl.BlockSpec((D, CLS), lambda i: (0, 0)),
        ],
        out_specs=pl.BlockSpec((BN, CLS), lambda i: (i, 0)),
        out_shape=jax.ShapeDtypeStruct((NPAD, CLS), jnp.float32),
    )(scat, g_flat, g_flat, dinv, b1r, W2)


def _scat2_body(g2_hbm, src_hbm, dst_hbm, out_hbm,
                srcv, dstv, r0, r1, acc, sem0, sem1):
    c = lax.axis_index("c")
    s = lax.axis_index("s")
    _zero_tile_share(r0, acc, s, CLS)
    pltpu.sync_copy(src_hbm.at[c, s], srcv)
    pltpu.sync_copy(dst_hbm.at[c, s], dstv)
    plsc.subcore_barrier()
    _edge_pass(g2_hbm, acc, srcv, dstv, r0, r1, sem0, sem1, CH2)
    plsc.subcore_barrier()
    pltpu.sync_copy(acc.at[pl.ds(s * RPT, RPT)],
                    out_hbm.at[c, pl.ds(s * RPT, RPT)])


def _scat2_call(g2, src2, dst2):
    return pl.kernel(
        _scat2_body,
        out_type=jax.ShapeDtypeStruct((NC, NPAD, CLS), jnp.float32),
        mesh=_mesh(),
        compiler_params=pltpu.CompilerParams(use_tc_tiling_on_sc=False),
        scratch_types=[
            pltpu.VMEM((CH2, CK), jnp.int32),
            pltpu.VMEM((CH2, CK), jnp.int32),
            pltpu.VMEM((CK, CLS), jnp.float32),
            pltpu.VMEM((CK, CLS), jnp.float32),
            pltpu.VMEM_SHARED((NACC, CLS), jnp.float32),
            pltpu.SemaphoreType.DMA,
            pltpu.SemaphoreType.DMA,
        ],
    )(g2, src2, dst2)


def _fin_body(p2_ref, g2_ref, dinv_ref, b2_ref, out_ref):
    agg = jnp.sum(p2_ref[...], axis=0) + g2_ref[...]
    out_ref[...] = agg * dinv_ref[...] + b2_ref[...]


FBN = 1000


def _fin_call(p2, g2, dinv, b2r):
    return pl.pallas_call(
        _fin_body,
        grid=(N // FBN,),
        in_specs=[
            pl.BlockSpec((NC, FBN, CLS), lambda i: (0, i, 0)),
            pl.BlockSpec((FBN, CLS), lambda i: (i, 0)),
            pl.BlockSpec((FBN, 1), lambda i: (i, 0)),
            pl.BlockSpec((1, CLS), lambda i: (0, 0)),
        ],
        out_specs=pl.BlockSpec((FBN, CLS), lambda i: (i, 0)),
        out_shape=jax.ShapeDtypeStruct((N, CLS), jnp.float32),
    )(p2, g2, dinv, b2r)


def kernel(x, edge_index, W1, b1, W2, b2):
    src = edge_index[0].astype(jnp.int32)
    dst = edge_index[1].astype(jnp.int32)
    pad_e = EPAD - E
    src_p = jnp.concatenate([src, jnp.zeros((pad_e,), jnp.int32)])
    dst_p = jnp.concatenate([dst, jnp.full((pad_e,), N, jnp.int32)])
    src1 = src_p.reshape(NS, CH1, CK)
    src1o = jnp.stack([src1, src1 + NPAD])
    dst1 = dst_p.reshape(NS, CH1, CK)
    src2 = src_p.reshape(NC, NS, CH2, CK)
    dst2 = dst_p.reshape(NC, NS, CH2, CK)
    x_pad = jnp.pad(x, ((0, NPAD - N), (0, 0)))
    b1r = b1.reshape(1, D)
    b2r = b2.reshape(1, CLS)

    degp = _deg_call(dst2)
    dinv = lax.rsqrt(degp[0] + degp[1] + 1.0).reshape(NPAD, 1)
    g3 = _g1_call(x_pad, W1, dinv)
    g_flat = g3.reshape(NC * NPAD, HALF)
    scat = _scat1_call(g_flat, src1o, dst1)
    g2 = _h2_call(scat, g_flat, dinv, b1r, W2)
    p2 = _scat2_call(g2, src2, dst2)
    return _fin_call(p2, g2, dinv, b2r)


# speedup vs baseline: 2.1494x; 1.1493x over previous
import jax
import jax.numpy as jnp
from jax import lax
from jax.experimental import pallas as pl
from jax.experimental.pallas import tpu as pltpu
from jax.experimental.pallas import tpu_sc as plsc

N = 10000
E = 160000
D = 256
HALF = 128
CLS = 16
NC = 2
NS = 16
L = 16

NPAD = 10240
CK = 128
CH1 = 80
EPAD = NS * CH1 * CK
CH2 = EPAD // (NC * NS) // CK
NACC = 10240
RPT = NACC // NS

BN = 1024
NB = NPAD // BN


def _mesh():
    return plsc.VectorSubcoreMesh(core_axis_name="c", subcore_axis_name="s")


def _deg_body(dst_hbm, degp_hbm, idxv, dloc2):
    c = lax.axis_index("c")
    s = lax.axis_index("s")
    pltpu.sync_copy(dst_hbm.at[c, s], idxv)
    zeros16 = jnp.zeros((L,), jnp.float32)

    def zero_body(i, carry):
        dloc2[i // 8, pl.ds((i % 8) * L, L)] = zeros16
        return carry

    lax.fori_loop(0, NPAD // L, zero_body, 0)
    ones16 = jnp.ones((L,), jnp.float32)
    KV = CK // L

    def hist_body(i, carry):
        j = i // KV
        k = i % KV
        idx = idxv[j, pl.ds(k * L, L)]
        plsc.addupdate_scatter(dloc2, [idx >> 7, idx & 127], ones16)
        return carry

    lax.fori_loop(0, CH2 * KV, hist_body, 0)
    pltpu.sync_copy(dloc2, degp_hbm.at[c, s])


def _deg_call(dst2):
    return pl.kernel(
        _deg_body,
        out_type=jax.ShapeDtypeStruct((NC, NS, NPAD // 128, 128), jnp.float32),
        mesh=_mesh(),
        compiler_params=pltpu.CompilerParams(needs_layout_passes=False),
        scratch_types=[
            pltpu.VMEM((CH2, CK), jnp.int32),
            pltpu.VMEM((NPAD // 128, 128), jnp.float32),
        ],
    )(dst2)


def _g1_body(x_ref, w1_ref, dinv_ref, g_ref):
    dinv = dinv_ref[...]
    h = jnp.dot(x_ref[...].astype(jnp.bfloat16), w1_ref[...].astype(jnp.bfloat16),
                preferred_element_type=jnp.float32)
    g = h * dinv
    g_ref[0] = g[:, :HALF]
    g_ref[1] = g[:, HALF:]


def _g1_call(x_pad, W1, dinv):
    return pl.pallas_call(
        _g1_body,
        grid=(NB,),
        in_specs=[
            pl.BlockSpec((BN, D), lambda i: (i, 0)),
            pl.BlockSpec((D, D), lambda i: (0, 0)),
            pl.BlockSpec((BN, 1), lambda i: (i, 0)),
        ],
        out_specs=pl.BlockSpec((NC, BN, HALF), lambda i: (0, i, 0)),
        out_shape=jax.ShapeDtypeStruct((NC, NPAD, HALF), jnp.float32),
    )(x_pad, W1, dinv)


GROUP = 16
NGRP = CH1 // GROUP


def _zero_tile_share(buf, acc, s, width):
    zeros16 = jnp.zeros((L,), jnp.float32)
    kv = width // L

    def zb(i, carry):
        buf[i // kv, pl.ds((i % kv) * L, L)] = zeros16
        return carry

    lax.fori_loop(0, CK * kv, zb, 0)
    for q in range(RPT // CK):
        pltpu.sync_copy(buf, acc.at[pl.ds(s * RPT + q * CK, CK)])


def _edge_pass(g_hbm, acc, srcv, dstv, r0, r1, sem0, sem1, nch):
    pltpu.async_copy(g_hbm.at[srcv.at[0]], r0, sem0)

    def step(j2, carry):
        j = j2 * 2
        pltpu.make_async_copy(g_hbm.at[srcv.at[j]], r0, sem0).wait()
        pltpu.async_copy(g_hbm.at[srcv.at[j + 1]], r1, sem1)
        pltpu.sync_copy(r0, acc.at[dstv.at[j]], add=True)
        pltpu.make_async_copy(g_hbm.at[srcv.at[j + 1]], r1, sem1).wait()

        @pl.when(j + 2 < nch)
        def _():
            pltpu.async_copy(g_hbm.at[srcv.at[j + 2]], r0, sem0)

        pltpu.sync_copy(r1, acc.at[dstv.at[j + 1]], add=True)
        return carry

    lax.fori_loop(0, nch // 2, step, 0)


def _scat1_body(g_hbm, src_hbm, dst_hbm, out_hbm,
                srcv, dstv, r0, r1, acc, sem0, sem1):
    c = lax.axis_index("c")
    s = lax.axis_index("s")
    _zero_tile_share(r0, acc, s, HALF)
    plsc.subcore_barrier()

    def group(gi, carry):
        pltpu.sync_copy(src_hbm.at[c, s, pl.ds(gi * GROUP, GROUP)], srcv)
        pltpu.sync_copy(dst_hbm.at[s, pl.ds(gi * GROUP, GROUP)], dstv)
        _edge_pass(g_hbm, acc, srcv, dstv, r0, r1, sem0, sem1, GROUP)
        return carry

    lax.fori_loop(0, NGRP, group, 0)
    plsc.subcore_barrier()
    pltpu.sync_copy(acc.at[pl.ds(s * RPT, RPT)],
                    out_hbm.at[c, pl.ds(s * RPT, RPT)])


def _scat1_call(g_flat, src1o, dst1):
    return pl.kernel(
        _scat1_body,
        out_type=jax.ShapeDtypeStruct((NC, NPAD, HALF), jnp.float32),
        mesh=_mesh(),
        scratch_types=[
            pltpu.VMEM((GROUP, CK), jnp.int32),
            pltpu.VMEM((GROUP, CK), jnp.int32),
            pltpu.VMEM((CK, HALF), jnp.float32),
            pltpu.VMEM((CK, HALF), jnp.float32),
            pltpu.VMEM_SHARED((NACC, HALF), jnp.float32),
            pltpu.SemaphoreType.DMA,
            pltpu.SemaphoreType.DMA,
        ],
    )(g_flat, src1o, dst1)


def _h2_body(scat_ref, ga_ref, gb_ref, dinv_ref, b1_ref, w2_ref, g2_ref):
    sc = scat_ref[...]
    h = jnp.concatenate([sc[0] + ga_ref[...], sc[1] + gb_ref[...]], axis=1)
    dinv = dinv_ref[...]
    o1 = jnp.maximum(h * dinv + b1_ref[...], 0.0)
    g2_ref[...] = jnp.dot(o1, w2_ref[...], preferred_element_type=jnp.float32) * dinv


def _h2_call(scat, g_flat, dinv, b1r, W2):
    return pl.pallas_call(
        _h2_body,
        grid=(NB,),
        in_specs=[
            pl.BlockSpec((NC, BN, HALF), lambda i: (0, i, 0)),
            pl.BlockSpec((BN, HALF), lambda i: (i, 0)),
            pl.BlockSpec((BN, HALF), lambda i: (NB + i, 0)),
            pl.BlockSpec((BN, 1), lambda i: (i, 0)),
            pl.BlockSpec((1, D), lambda i: (0, 0)),
            pl.BlockSpec((D, CLS), lambda i: (0, 0)),
        ],
        out_specs=pl.BlockSpec((BN, CLS), lambda i: (i, 0)),
        out_shape=jax.ShapeDtypeStruct((NPAD, CLS), jnp.float32),
    )(scat, g_flat, g_flat, dinv, b1r, W2)


def _scat2_body(g2_hbm, src_hbm, dst_hbm, out_hbm,
                srcv, dstv, r0, r1, acc, sem0, sem1):
    c = lax.axis_index("c")
    s = lax.axis_index("s")
    _zero_tile_share(r0, acc, s, CLS)
    pltpu.sync_copy(src_hbm.at[c, s], srcv)
    pltpu.sync_copy(dst_hbm.at[c, s], dstv)
    plsc.subcore_barrier()
    _edge_pass(g2_hbm, acc, srcv, dstv, r0, r1, sem0, sem1, CH2)
    plsc.subcore_barrier()
    pltpu.sync_copy(acc.at[pl.ds(s * RPT, RPT)],
                    out_hbm.at[c, pl.ds(s * RPT, RPT)])


def _scat2_call(g2, src2, dst2):
    return pl.kernel(
        _scat2_body,
        out_type=jax.ShapeDtypeStruct((NC, NPAD, CLS), jnp.float32),
        mesh=_mesh(),
        compiler_params=pltpu.CompilerParams(use_tc_tiling_on_sc=False),
        scratch_types=[
            pltpu.VMEM((CH2, CK), jnp.int32),
            pltpu.VMEM((CH2, CK), jnp.int32),
            pltpu.VMEM((CK, CLS), jnp.float32),
            pltpu.VMEM((CK, CLS), jnp.float32),
            pltpu.VMEM_SHARED((NACC, CLS), jnp.float32),
            pltpu.SemaphoreType.DMA,
            pltpu.SemaphoreType.DMA,
        ],
    )(g2, src2, dst2)


def _fin_body(p2_ref, g2_ref, dinv_ref, b2_ref, out_ref):
    agg = jnp.sum(p2_ref[...], axis=0) + g2_ref[...]
    out_ref[...] = agg * dinv_ref[...] + b2_ref[...]


FBN = 1000


def _fin_call(p2, g2, dinv, b2r):
    return pl.pallas_call(
        _fin_body,
        grid=(N // FBN,),
        in_specs=[
            pl.BlockSpec((NC, FBN, CLS), lambda i: (0, i, 0)),
            pl.BlockSpec((FBN, CLS), lambda i: (i, 0)),
            pl.BlockSpec((FBN, 1), lambda i: (i, 0)),
            pl.BlockSpec((1, CLS), lambda i: (0, 0)),
        ],
        out_specs=pl.BlockSpec((FBN, CLS), lambda i: (i, 0)),
        out_shape=jax.ShapeDtypeStruct((N, CLS), jnp.float32),
    )(p2, g2, dinv, b2r)


def kernel(x, edge_index, W1, b1, W2, b2):
    src = edge_index[0].astype(jnp.int32)
    dst = edge_index[1].astype(jnp.int32)
    pad_e = EPAD - E
    src_p = jnp.concatenate([src, jnp.zeros((pad_e,), jnp.int32)])
    dst_p = jnp.concatenate([dst, jnp.full((pad_e,), N, jnp.int32)])
    src1 = src_p.reshape(NS, CH1, CK)
    src1o = jnp.stack([src1, src1 + NPAD])
    dst1 = dst_p.reshape(NS, CH1, CK)
    src2 = src_p.reshape(NC, NS, CH2, CK)
    dst2 = dst_p.reshape(NC, NS, CH2, CK)
    x_pad = jnp.pad(x, ((0, NPAD - N), (0, 0)))
    b1r = b1.reshape(1, D)
    b2r = b2.reshape(1, CLS)

    degp = _deg_call(dst2)
    dinv = lax.rsqrt(jnp.sum(degp, axis=(0, 1)) + 1.0).reshape(NPAD, 1)
    g3 = _g1_call(x_pad, W1, dinv)
    g_flat = g3.reshape(NC * NPAD, HALF)
    scat = _scat1_call(g_flat, src1o, dst1)
    g2 = _h2_call(scat, g_flat, dinv, b1r, W2)
    p2 = _scat2_call(g2, src2, dst2)
    return _fin_call(p2, g2, dinv, b2r)
